# baseline (speedup 1.0000x reference)
import jax
import jax.numpy as jnp
from jax.experimental import pallas as pl
from jax.experimental.pallas import tpu as pltpu

_PH, _PW = 7, 7
_SCALE = 0.0625
_NLVL = 4
_G = 8


def _roi_pool_body(rkb, rlo, rhi, cix, perm, feat, out, t, rm, mm, sem):
    r = pl.program_id(0)
    B = feat.shape[0]
    H = feat.shape[1]
    W = feat.shape[2]
    C = feat.shape[3]

    @pl.when(r == 0)
    def _init():
        mm[...] = jnp.zeros_like(mm)
        cp = pltpu.make_async_copy(feat, t.at[0:B], sem)
        cp.start()
        cp.wait()
        for l in range(1, _NLVL):
            d = 1 << (l - 1)
            for b_ in range(B):
                src = (l - 1) * B + b_
                dst = l * B + b_
                t[dst, 0:H - d] = jnp.maximum(t[src, 0:H - d], t[src, d:H])
                t[dst, H - d:H] = jnp.maximum(
                    t[src, H - d:H],
                    jnp.broadcast_to(t[src, H - 1:H], (d, W, C)),
                )

    for g in range(_G):
        base7 = (r * _G + g) * _PH
        for i in range(_PH):
            kb = rkb[base7 + i]
            rm[g, i] = jnp.maximum(t[kb, rlo[base7 + i]], t[kb, rhi[base7 + i]])

    for g in range(_G):
        base9 = (r * _G + g) * (_PW * 9)
        for i in range(_PH):
            o = base9 + i * 9
            acc = rm[g, :, cix[o], :]
            for k in range(1, 9):
                acc = jnp.maximum(acc, rm[g, :, cix[o + k], :])
            mm[g, i * 8:i * 8 + _PW] = acc
    for g in range(_G):
        out[g] = jax.lax.dot_general(
            mm[g], perm[...],
            (((0,), (0,)), ((), ())),
            preferred_element_type=jnp.float32,
        )


def kernel(features, rois):
    B, C, H, W = features.shape
    R = rois.shape[0]
    r32 = rois.astype(jnp.int32)
    coords = jnp.floor(r32[:, 1:].astype(jnp.float32) * _SCALE).astype(jnp.int32)
    b = r32[:, 0]
    x1, y1, x2, y2 = coords[:, 0], coords[:, 1], coords[:, 2], coords[:, 3]
    h = y2 - y1 + 1
    w = x2 - x1 + 1

    i7 = jnp.arange(_PH, dtype=jnp.int32)
    sw = jnp.minimum(x1[:, None] + (i7 * w[:, None]) // _PW, W - 1)
    ew = jnp.minimum(x1[:, None] + ((i7 + 1) * w[:, None] + _PW - 1) // _PW, W)
    nw = ew - sw
    kw = ((nw > 1).astype(jnp.int32) + (nw > 3).astype(jnp.int32)
          + (nw > 7).astype(jnp.int32))
    rkb = (kw * B + b[:, None]).reshape(-1)
    rlo = sw.reshape(-1)
    rhi = (ew - jnp.left_shift(jnp.int32(1), kw)).reshape(-1)

    sh = jnp.minimum(y1[:, None] + (i7 * h[:, None]) // _PH, H - 1)
    eh = jnp.minimum(y1[:, None] + ((i7 + 1) * h[:, None] + _PH - 1) // _PH, H)
    k9 = jnp.arange(9, dtype=jnp.int32)
    cix = jnp.minimum(sh[:, :, None] + k9, (eh - 1)[:, :, None]).reshape(-1)

    feat = jnp.transpose(features, (0, 3, 2, 1))

    q56 = jnp.arange(8 * _PH, dtype=jnp.int32)
    t49 = (jnp.arange(_PH * _PW, dtype=jnp.int32) // _PW) * 8 + (
        jnp.arange(_PH * _PW, dtype=jnp.int32) % _PW)
    perm = (q56[:, None] == t49[None, :]).astype(jnp.float32)

    smem = pl.BlockSpec(memory_space=pltpu.SMEM)
    out = pl.pallas_call(
        _roi_pool_body,
        grid=(R // _G,),
        in_specs=[
            smem, smem, smem, smem,
            pl.BlockSpec((8 * _PH, _PH * _PW), lambda r: (0, 0)),
            pl.BlockSpec(memory_space=pl.ANY),
        ],
        out_specs=pl.BlockSpec((_G, C, _PW * _PH), lambda r: (r, 0, 0)),
        out_shape=jax.ShapeDtypeStruct((R, C, _PW * _PH), jnp.float32),
        scratch_shapes=[
            pltpu.VMEM((_NLVL * B, H, W, C), jnp.float32),
            pltpu.VMEM((_G, _PH, W, C), jnp.float32),
            pltpu.VMEM((_G, 8 * _PH, C), jnp.float32),
            pltpu.SemaphoreType.DMA,
        ],
        compiler_params=pltpu.CompilerParams(
            dimension_semantics=("arbitrary",),
            vmem_limit_bytes=50 * 1024 * 1024,
        ),
    )(rkb, rlo, rhi, cix, perm, feat)

    return out.reshape(R, C, _PH, _PW)


# speedup vs baseline: 1.0973x; 1.0973x over previous
import jax
import jax.numpy as jnp
from jax.experimental import pallas as pl
from jax.experimental.pallas import tpu as pltpu

_PH, _PW = 7, 7
_SCALE = 0.0625
_NLVL = 4
_G = 8


def _roi_pool_body(rkb, rlo, rhi, cix, feat, out, t, rm, sbuf, sem):
    r = pl.program_id(0)
    B = feat.shape[0]
    C = feat.shape[1]
    H = t.shape[1]
    W = t.shape[2]

    @pl.when(r == 0)
    def _init():
        for b_ in range(B):
            cp = pltpu.make_async_copy(feat.at[b_], sbuf, sem)
            cp.start()
            cp.wait()
            t[b_] = jnp.transpose(sbuf[...]).reshape(H, W, C)
        for l in range(1, _NLVL):
            d = 1 << (l - 1)
            for b_ in range(B):
                src = (l - 1) * B + b_
                dst = l * B + b_
                t[dst, 0:H - d] = jnp.maximum(t[src, 0:H - d], t[src, d:H])
                t[dst, H - d:H] = jnp.maximum(
                    t[src, H - d:H],
                    jnp.broadcast_to(t[src, H - 1:H], (d, W, C)),
                )

    for g in range(_G):
        base7 = (r * _G + g) * _PH
        for i in range(_PH):
            kb = rkb[base7 + i]
            rm[g, i] = jnp.maximum(t[kb, rlo[base7 + i]], t[kb, rhi[base7 + i]])

    for g in range(_G):
        base9 = (r * _G + g) * (_PW * 9)
        for j in range(_PW):
            o = base9 + j * 9
            acc = rm[g, :, cix[o], :]
            for k in range(1, 9):
                acc = jnp.maximum(acc, rm[g, :, cix[o + k], :])
            out[g, j] = acc


def kernel(features, rois):
    B, C, H, W = features.shape
    R = rois.shape[0]
    r32 = rois.astype(jnp.int32)
    coords = jnp.floor(r32[:, 1:].astype(jnp.float32) * _SCALE).astype(jnp.int32)
    b = r32[:, 0]
    x1, y1, x2, y2 = coords[:, 0], coords[:, 1], coords[:, 2], coords[:, 3]
    h = y2 - y1 + 1
    w = x2 - x1 + 1

    i7 = jnp.arange(_PH, dtype=jnp.int32)
    sh = jnp.minimum(y1[:, None] + (i7 * h[:, None]) // _PH, H - 1)
    eh = jnp.minimum(y1[:, None] + ((i7 + 1) * h[:, None] + _PH - 1) // _PH, H)
    nh = eh - sh
    kh = ((nh > 1).astype(jnp.int32) + (nh > 3).astype(jnp.int32)
          + (nh > 7).astype(jnp.int32))
    rkb = (kh * B + b[:, None]).reshape(-1)
    rlo = sh.reshape(-1)
    rhi = (eh - jnp.left_shift(jnp.int32(1), kh)).reshape(-1)

    sw = jnp.minimum(x1[:, None] + (i7 * w[:, None]) // _PW, W - 1)
    ew = jnp.minimum(x1[:, None] + ((i7 + 1) * w[:, None] + _PW - 1) // _PW, W)
    k9 = jnp.arange(9, dtype=jnp.int32)
    cix = jnp.minimum(sw[:, :, None] + k9, (ew - 1)[:, :, None]).reshape(-1)

    featflat = features.reshape(B, C, H * W)

    smem = pl.BlockSpec(memory_space=pltpu.SMEM)
    out = pl.pallas_call(
        _roi_pool_body,
        grid=(R // _G,),
        in_specs=[
            smem, smem, smem, smem,
            pl.BlockSpec(memory_space=pl.ANY),
        ],
        out_specs=pl.BlockSpec((_G, _PW, _PH, C), lambda r: (r, 0, 0, 0)),
        out_shape=jax.ShapeDtypeStruct((R, _PW, _PH, C), jnp.float32),
        scratch_shapes=[
            pltpu.VMEM((_NLVL * B, H, W, C), jnp.float32),
            pltpu.VMEM((_G, _PH, W, C), jnp.float32),
            pltpu.VMEM((C, H * W), jnp.float32),
            pltpu.SemaphoreType.DMA,
        ],
        compiler_params=pltpu.CompilerParams(
            dimension_semantics=("arbitrary",),
            vmem_limit_bytes=52 * 1024 * 1024,
        ),
    )(rkb, rlo, rhi, cix, featflat)

    return jnp.transpose(out, (0, 3, 2, 1))


# speedup vs baseline: 1.3685x; 1.2471x over previous
import jax
import jax.numpy as jnp
from jax.experimental import pallas as pl
from jax.experimental.pallas import tpu as pltpu

_PH, _PW = 7, 7
_SCALE = 0.0625
_NLVL = 4
_G = 8


def _roi_pool_body(rkb, rlo, rhi, cix, nmx, feat, out, t, rm, sem):
    r = pl.program_id(0)
    B = feat.shape[0]
    H = feat.shape[1]
    W = feat.shape[2]
    C = feat.shape[3]

    @pl.when(r == 0)
    def _init():
        cp = pltpu.make_async_copy(feat, t.at[0:B], sem)
        cp.start()
        cp.wait()
        for l in range(1, _NLVL):
            d = 1 << (l - 1)
            for b_ in range(B):
                src = (l - 1) * B + b_
                dst = l * B + b_
                t[dst, 0:H - d] = jnp.maximum(t[src, 0:H - d], t[src, d:H])
                t[dst, H - d:H] = jnp.maximum(
                    t[src, H - d:H],
                    jnp.broadcast_to(t[src, H - 1:H], (d, W, C)),
                )

    for g in range(_G):
        base7 = (r * _G + g) * _PH
        for i in range(_PH):
            kb = rkb[base7 + i]
            rm[g, i] = jnp.maximum(t[kb, rlo[base7 + i]], t[kb, rhi[base7 + i]])

    for g in range(_G):
        base9 = (r * _G + g) * (_PW * 9)
        nm = nmx[r * _G + g]

        def _cols(kmax, base9=base9, g=g):
            for j in range(_PW):
                o = base9 + j * 9
                acc = rm[g, :, cix[o], :]
                for k in range(1, kmax):
                    acc = jnp.maximum(acc, rm[g, :, cix[o + k], :])
                out[g, j] = acc

        pl.when(nm <= 4)(lambda: _cols(4))
        pl.when(jnp.logical_and(nm > 4, nm <= 6))(lambda: _cols(6))
        pl.when(nm > 6)(lambda: _cols(9))


def kernel(features, rois):
    B, C, H, W = features.shape
    R = rois.shape[0]
    r32 = rois.astype(jnp.int32)
    coords = jnp.floor(r32[:, 1:].astype(jnp.float32) * _SCALE).astype(jnp.int32)
    b = r32[:, 0]
    x1, y1, x2, y2 = coords[:, 0], coords[:, 1], coords[:, 2], coords[:, 3]
    h = y2 - y1 + 1
    w = x2 - x1 + 1

    i7 = jnp.arange(_PH, dtype=jnp.int32)
    sh = jnp.minimum(y1[:, None] + (i7 * h[:, None]) // _PH, H - 1)
    eh = jnp.minimum(y1[:, None] + ((i7 + 1) * h[:, None] + _PH - 1) // _PH, H)
    nh = eh - sh
    kh = ((nh > 1).astype(jnp.int32) + (nh > 3).astype(jnp.int32)
          + (nh > 7).astype(jnp.int32))
    rkb = (kh * B + b[:, None]).reshape(-1)
    rlo = sh.reshape(-1)
    rhi = (eh - jnp.left_shift(jnp.int32(1), kh)).reshape(-1)

    sw = jnp.minimum(x1[:, None] + (i7 * w[:, None]) // _PW, W - 1)
    ew = jnp.minimum(x1[:, None] + ((i7 + 1) * w[:, None] + _PW - 1) // _PW, W)
    k9 = jnp.arange(9, dtype=jnp.int32)
    cix = jnp.minimum(sw[:, :, None] + k9, (ew - 1)[:, :, None]).reshape(-1)
    nmx = jnp.max(ew - sw, axis=1)

    feat = jnp.transpose(features, (0, 2, 3, 1))

    smem = pl.BlockSpec(memory_space=pltpu.SMEM)
    out = pl.pallas_call(
        _roi_pool_body,
        grid=(R // _G,),
        in_specs=[
            smem, smem, smem, smem, smem,
            pl.BlockSpec(memory_space=pl.ANY),
        ],
        out_specs=pl.BlockSpec((_G, _PW, _PH, C), lambda r: (r, 0, 0, 0)),
        out_shape=jax.ShapeDtypeStruct((R, _PW, _PH, C), jnp.float32),
        scratch_shapes=[
            pltpu.VMEM((_NLVL * B, H, W, C), jnp.float32),
            pltpu.VMEM((_G, _PH, W, C), jnp.float32),
            pltpu.SemaphoreType.DMA,
        ],
        compiler_params=pltpu.CompilerParams(
            dimension_semantics=("arbitrary",),
            vmem_limit_bytes=50 * 1024 * 1024,
        ),
    )(rkb, rlo, rhi, cix, nmx, feat)

    return jnp.transpose(out, (0, 3, 2, 1))


# speedup vs baseline: 1.3929x; 1.0178x over previous
import jax
import jax.numpy as jnp
from jax.experimental import pallas as pl
from jax.experimental.pallas import tpu as pltpu

_PH, _PW = 7, 7
_SCALE = 0.0625
_NLVL = 4
_G = 8


def _roi_pool_body(rkb, rlo, rhi, cix, nmx, feat, out, t, rm, sem):
    r = pl.program_id(0)
    B = feat.shape[0]
    H = feat.shape[1]
    W = feat.shape[2]
    C = feat.shape[3]

    @pl.when(r == 0)
    def _init():
        cp = pltpu.make_async_copy(feat, t.at[0:B], sem)
        cp.start()
        cp.wait()
        for l in range(1, _NLVL):
            d = 1 << (l - 1)
            for b_ in range(B):
                src = (l - 1) * B + b_
                dst = l * B + b_
                t[dst, 0:H - d] = jnp.maximum(t[src, 0:H - d], t[src, d:H])
                t[dst, H - d:H] = jnp.maximum(
                    t[src, H - d:H],
                    jnp.broadcast_to(t[src, H - 1:H], (d, W, C)),
                )

    for g in range(_G):
        base7 = (r * _G + g) * _PH
        for i in range(_PH):
            kb = rkb[base7 + i]
            rm[g, i] = jnp.maximum(t[kb, rlo[base7 + i]], t[kb, rhi[base7 + i]])

    for g in range(_G):
        base9 = (r * _G + g) * (_PW * 9)
        nm = nmx[r * _G + g]

        def _cols(kmax, base9=base9, g=g):
            for j in range(_PW):
                o = base9 + j * 9
                acc = rm[g, :, cix[o], :]
                for k in range(1, kmax):
                    acc = jnp.maximum(acc, rm[g, :, cix[o + k], :])
                out[g, j] = acc

        pl.when(nm <= 2)(lambda: _cols(2))
        pl.when(jnp.logical_and(nm > 2, nm <= 4))(lambda: _cols(4))
        pl.when(jnp.logical_and(nm > 4, nm <= 6))(lambda: _cols(6))
        pl.when(nm > 6)(lambda: _cols(9))


def kernel(features, rois):
    B, C, H, W = features.shape
    R = rois.shape[0]
    r32 = rois.astype(jnp.int32)
    coords = jnp.floor(r32[:, 1:].astype(jnp.float32) * _SCALE).astype(jnp.int32)
    b = r32[:, 0]
    x1, y1, x2, y2 = coords[:, 0], coords[:, 1], coords[:, 2], coords[:, 3]
    h = y2 - y1 + 1
    w = x2 - x1 + 1

    i7 = jnp.arange(_PH, dtype=jnp.int32)
    sh = jnp.minimum(y1[:, None] + (i7 * h[:, None]) // _PH, H - 1)
    eh = jnp.minimum(y1[:, None] + ((i7 + 1) * h[:, None] + _PH - 1) // _PH, H)
    nh = eh - sh
    kh = ((nh > 1).astype(jnp.int32) + (nh > 3).astype(jnp.int32)
          + (nh > 7).astype(jnp.int32))
    rkb = (kh * B + b[:, None]).reshape(-1)
    rlo = sh.reshape(-1)
    rhi = (eh - jnp.left_shift(jnp.int32(1), kh)).reshape(-1)

    sw = jnp.minimum(x1[:, None] + (i7 * w[:, None]) // _PW, W - 1)
    ew = jnp.minimum(x1[:, None] + ((i7 + 1) * w[:, None] + _PW - 1) // _PW, W)
    k9 = jnp.arange(9, dtype=jnp.int32)
    cix = jnp.minimum(sw[:, :, None] + k9, (ew - 1)[:, :, None]).reshape(-1)
    nmx = jnp.max(ew - sw, axis=1)

    feat = jnp.transpose(features, (0, 2, 3, 1))

    smem = pl.BlockSpec(memory_space=pltpu.SMEM)
    out = pl.pallas_call(
        _roi_pool_body,
        grid=(R // _G,),
        in_specs=[
            smem, smem, smem, smem, smem,
            pl.BlockSpec(memory_space=pl.ANY),
        ],
        out_specs=pl.BlockSpec((_G, _PW, _PH, C), lambda r: (r, 0, 0, 0)),
        out_shape=jax.ShapeDtypeStruct((R, _PW, _PH, C), jnp.float32),
        scratch_shapes=[
            pltpu.VMEM((_NLVL * B, H, W, C), jnp.float32),
            pltpu.VMEM((_G, _PH, W, C), jnp.float32),
            pltpu.SemaphoreType.DMA,
        ],
        compiler_params=pltpu.CompilerParams(
            dimension_semantics=("arbitrary",),
            vmem_limit_bytes=50 * 1024 * 1024,
        ),
    )(rkb, rlo, rhi, cix, nmx, feat)

    return jnp.transpose(out, (0, 3, 2, 1))


# speedup vs baseline: 1.3960x; 1.0022x over previous
import jax
import jax.numpy as jnp
from jax.experimental import pallas as pl
from jax.experimental.pallas import tpu as pltpu

_PH, _PW = 7, 7
_SCALE = 0.0625
_NLVL = 4
_G = 16


def _roi_pool_body(rkb, rlo, rhi, cix, nmx, feat, out, t, rm, sem):
    r = pl.program_id(0)
    B = feat.shape[0]
    H = feat.shape[1]
    W = feat.shape[2]
    C = feat.shape[3]

    @pl.when(r == 0)
    def _init():
        cp = pltpu.make_async_copy(feat, t.at[0:B], sem)
        cp.start()
        cp.wait()
        for l in range(1, _NLVL):
            d = 1 << (l - 1)
            for b_ in range(B):
                src = (l - 1) * B + b_
                dst = l * B + b_
                t[dst, 0:H - d] = jnp.maximum(t[src, 0:H - d], t[src, d:H])
                t[dst, H - d:H] = jnp.maximum(
                    t[src, H - d:H],
                    jnp.broadcast_to(t[src, H - 1:H], (d, W, C)),
                )

    for g in range(_G):
        base7 = (r * _G + g) * _PH
        for i in range(_PH):
            kb = rkb[base7 + i]
            rm[g, i] = jnp.maximum(t[kb, rlo[base7 + i]], t[kb, rhi[base7 + i]])

    for g in range(_G):
        base9 = (r * _G + g) * (_PW * 9)
        nm = nmx[r * _G + g]

        def _cols(kmax, base9=base9, g=g):
            for j in range(_PW):
                o = base9 + j * 9
                acc = rm[g, :, cix[o], :]
                for k in range(1, kmax):
                    acc = jnp.maximum(acc, rm[g, :, cix[o + k], :])
                out[g, j] = acc

        pl.when(nm <= 2)(lambda: _cols(2))
        pl.when(jnp.logical_and(nm > 2, nm <= 4))(lambda: _cols(4))
        pl.when(jnp.logical_and(nm > 4, nm <= 6))(lambda: _cols(6))
        pl.when(nm > 6)(lambda: _cols(9))


def kernel(features, rois):
    B, C, H, W = features.shape
    R = rois.shape[0]
    r32 = rois.astype(jnp.int32)
    coords = jnp.floor(r32[:, 1:].astype(jnp.float32) * _SCALE).astype(jnp.int32)
    b = r32[:, 0]
    x1, y1, x2, y2 = coords[:, 0], coords[:, 1], coords[:, 2], coords[:, 3]
    h = y2 - y1 + 1
    w = x2 - x1 + 1

    i7 = jnp.arange(_PH, dtype=jnp.int32)
    sh = jnp.minimum(y1[:, None] + (i7 * h[:, None]) // _PH, H - 1)
    eh = jnp.minimum(y1[:, None] + ((i7 + 1) * h[:, None] + _PH - 1) // _PH, H)
    nh = eh - sh
    kh = ((nh > 1).astype(jnp.int32) + (nh > 3).astype(jnp.int32)
          + (nh > 7).astype(jnp.int32))
    rkb = (kh * B + b[:, None]).reshape(-1)
    rlo = sh.reshape(-1)
    rhi = (eh - jnp.left_shift(jnp.int32(1), kh)).reshape(-1)

    sw = jnp.minimum(x1[:, None] + (i7 * w[:, None]) // _PW, W - 1)
    ew = jnp.minimum(x1[:, None] + ((i7 + 1) * w[:, None] + _PW - 1) // _PW, W)
    k9 = jnp.arange(9, dtype=jnp.int32)
    cix = jnp.minimum(sw[:, :, None] + k9, (ew - 1)[:, :, None]).reshape(-1)
    nmx = jnp.max(ew - sw, axis=1)

    feat = jnp.transpose(features, (0, 2, 3, 1))

    smem = pl.BlockSpec(memory_space=pltpu.SMEM)
    out = pl.pallas_call(
        _roi_pool_body,
        grid=(R // _G,),
        in_specs=[
            smem, smem, smem, smem, smem,
            pl.BlockSpec(memory_space=pl.ANY),
        ],
        out_specs=pl.BlockSpec((_G, _PW, _PH, C), lambda r: (r, 0, 0, 0)),
        out_shape=jax.ShapeDtypeStruct((R, _PW, _PH, C), jnp.float32),
        scratch_shapes=[
            pltpu.VMEM((_NLVL * B, H, W, C), jnp.float32),
            pltpu.VMEM((_G, _PH, W, C), jnp.float32),
            pltpu.SemaphoreType.DMA,
        ],
        compiler_params=pltpu.CompilerParams(
            dimension_semantics=("arbitrary",),
            vmem_limit_bytes=56 * 1024 * 1024,
        ),
    )(rkb, rlo, rhi, cix, nmx, feat)

    return jnp.transpose(out, (0, 3, 2, 1))


# speedup vs baseline: 1.3973x; 1.0009x over previous
import jax
import jax.numpy as jnp
from jax.experimental import pallas as pl
from jax.experimental.pallas import tpu as pltpu

_PH, _PW = 7, 7
_SCALE = 0.0625
_NLVL = 4
_G = 8


def _roi_pool_body(rkb, rlo, rhi, cix, nmx, feat, out, t, rm, sem):
    r = pl.program_id(0)
    B = feat.shape[0]
    H = feat.shape[1]
    W = feat.shape[2]
    C = feat.shape[3]

    @pl.when(r == 0)
    def _init():
        cp = pltpu.make_async_copy(feat, t.at[0:B], sem)
        cp.start()
        cp.wait()
        for l in range(1, _NLVL):
            d = 1 << (l - 1)
            for b_ in range(B):
                src = (l - 1) * B + b_
                dst = l * B + b_
                t[dst, 0:H - d] = jnp.maximum(t[src, 0:H - d], t[src, d:H])
                t[dst, H - d:H] = jnp.maximum(
                    t[src, H - d:H],
                    jnp.broadcast_to(t[src, H - 1:H], (d, W, C)),
                )

    for g in range(_G):
        base7 = (r * _G + g) * _PH
        for i in range(_PH):
            kb = rkb[base7 + i]
            rm[g, i] = jnp.maximum(t[kb, rlo[base7 + i]], t[kb, rhi[base7 + i]])

    for g in range(_G):
        base9 = (r * _G + g) * (_PW * 9)
        nm = nmx[r * _G + g]

        def _cols(kmax, base9=base9, g=g):
            for j in range(_PW):
                o = base9 + j * 9
                acc = rm[g, :, cix[o], :]
                for k in range(1, kmax):
                    acc = jnp.maximum(acc, rm[g, :, cix[o + k], :])
                out[g, j] = acc

        pl.when(nm <= 2)(lambda: _cols(2))
        pl.when(jnp.logical_and(nm > 2, nm <= 4))(lambda: _cols(4))
        pl.when(jnp.logical_and(nm > 4, nm <= 6))(lambda: _cols(6))
        pl.when(nm > 6)(lambda: _cols(9))


def kernel(features, rois):
    B, C, H, W = features.shape
    R = rois.shape[0]
    r32 = rois.astype(jnp.int32)
    coords = jnp.floor(r32[:, 1:].astype(jnp.float32) * _SCALE).astype(jnp.int32)
    b = r32[:, 0]
    x1, y1, x2, y2 = coords[:, 0], coords[:, 1], coords[:, 2], coords[:, 3]
    h = y2 - y1 + 1
    w = x2 - x1 + 1

    i7 = jnp.arange(_PH, dtype=jnp.int32)
    sh = jnp.minimum(y1[:, None] + (i7 * h[:, None]) // _PH, H - 1)
    eh = jnp.minimum(y1[:, None] + ((i7 + 1) * h[:, None] + _PH - 1) // _PH, H)
    nh = eh - sh
    kh = ((nh > 1).astype(jnp.int32) + (nh > 3).astype(jnp.int32)
          + (nh > 7).astype(jnp.int32))
    rkb = (kh * B + b[:, None]).reshape(-1)
    rlo = sh.reshape(-1)
    rhi = (eh - jnp.left_shift(jnp.int32(1), kh)).reshape(-1)

    sw = jnp.minimum(x1[:, None] + (i7 * w[:, None]) // _PW, W - 1)
    ew = jnp.minimum(x1[:, None] + ((i7 + 1) * w[:, None] + _PW - 1) // _PW, W)
    k9 = jnp.arange(9, dtype=jnp.int32)
    cix = jnp.minimum(sw[:, :, None] + k9, (ew - 1)[:, :, None]).reshape(-1)
    nmx = jnp.max(ew - sw, axis=1)

    feat = jnp.transpose(features, (0, 2, 3, 1))

    smem = pl.BlockSpec(memory_space=pltpu.SMEM)
    out = pl.pallas_call(
        _roi_pool_body,
        grid=(R // _G,),
        in_specs=[
            smem, smem, smem, smem, smem,
            pl.BlockSpec(memory_space=pl.ANY),
        ],
        out_specs=pl.BlockSpec((_G, _PW, _PH, C), lambda r: (r, 0, 0, 0)),
        out_shape=jax.ShapeDtypeStruct((R, _PW, _PH, C), jnp.float32),
        scratch_shapes=[
            pltpu.VMEM((_NLVL * B, H, W, C), jnp.float32),
            pltpu.VMEM((_G, _PH, W, C), jnp.float32),
            pltpu.SemaphoreType.DMA,
        ],
        compiler_params=pltpu.CompilerParams(
            dimension_semantics=("arbitrary",),
            vmem_limit_bytes=50 * 1024 * 1024,
        ),
    )(rkb, rlo, rhi, cix, nmx, feat)

    return jnp.transpose(out, (0, 3, 2, 1))
